# baseline (speedup 1.0000x reference)
import jax
import jax.numpy as jnp
from jax import lax
from jax.experimental import pallas as pl
from jax.experimental.pallas import tpu as pltpu
from jax.experimental.pallas import tpu_sc as plsc

B, C, NP, R, COUT = 32, 128, 4096, 32, 256
N = B * NP
NSEG = B * R * R

NC, NS = 2, 16
NW = NC * NS

K1_CHUNK = 2048
K1_PER_TILE = N // NS
K1_NCH = K1_PER_TILE // K1_CHUNK


def _k1_body(ci_hbm, vals_hbm, z_hbm, sums_out, idxout_hbm,
             ci_v, idx_cur, vals_v, fidx_v, sums):
    c = lax.axis_index("c")
    s = lax.axis_index("s")
    pltpu.sync_copy(z_hbm, sums.at[pl.ds(s * (NSEG * 4 // NS), NSEG * 4 // NS)])
    plsc.subcore_barrier()

    iota = lax.iota(jnp.int32, 16)
    for t in range(K1_NCH):
        base = s * K1_PER_TILE + t * K1_CHUNK
        pltpu.sync_copy(ci_hbm.at[pl.ds(base * 3, K1_CHUNK * 3)], ci_v)
        pltpu.sync_copy(vals_hbm.at[pl.ds(base * 4, K1_CHUNK * 4)], vals_v)

        def body(i, carry):
            f = (i * 16 + iota) * 3
            c0 = plsc.load_gather(ci_v, [f])
            c1 = plsc.load_gather(ci_v, [f + 1])
            c2 = plsc.load_gather(ci_v, [f + 2])
            idxv = c0 * (R * R) + c1 * R + c2
            idx_cur[pl.ds(i * 16, 16)] = idxv
            return carry

        lax.fori_loop(0, K1_CHUNK // 16, body, 0)

        @pl.when(c == 0)
        def _():
            pltpu.sync_copy(idx_cur, idxout_hbm.at[pl.ds(base, K1_CHUNK)])

        def fbody(g, carry):
            pos = g * 16 + iota
            k = lax.shift_right_logical(pos, 2)
            comp = lax.bitwise_and(pos, 3)
            segs = plsc.load_gather(idx_cur, [k])
            fidx_v[pl.ds(g * 16, 16)] = segs * 4 + comp
            return carry

        lax.fori_loop(0, K1_CHUNK * 4 // 16, fbody, 0)
        pltpu.sync_copy(vals_v, sums.at[fidx_v], add=True)

    plsc.subcore_barrier()
    @pl.when(c == 0)
    def _():
        pltpu.sync_copy(sums.at[pl.ds(s * (NSEG * 4 // NS), NSEG * 4 // NS)],
                        sums_out.at[pl.ds(s * (NSEG * 4 // NS),
                                          NSEG * 4 // NS)])


def _k1(ci_flat, vals_flat, zflat):
    mesh = plsc.VectorSubcoreMesh(core_axis_name="c", subcore_axis_name="s",
                                  num_cores=NC, num_subcores=NS)
    f = pl.kernel(
        _k1_body,
        out_type=[jax.ShapeDtypeStruct((NSEG * 4,), jnp.float32),
                  jax.ShapeDtypeStruct((N,), jnp.int32)],
        mesh=mesh,
        scratch_types=[
            pltpu.VMEM((K1_CHUNK * 3,), jnp.int32),
            pltpu.VMEM((K1_CHUNK,), jnp.int32),
            pltpu.VMEM((K1_CHUNK * 4,), jnp.float32),
            pltpu.VMEM((K1_CHUNK * 4,), jnp.int32),
            pltpu.VMEM_SHARED((NSEG * 4,), jnp.float32),
        ],
        compiler_params=pltpu.CompilerParams(needs_layout_passes=False,
                                             use_tc_tiling_on_sc=False),
    )
    return f(ci_flat, vals_flat, zflat)


BP = 512


def _k2_body(ft_ref, nc_ref, pv_ref, wf_ref, wn_ref, sl_ref,
             o0_ref, o1_ref):
    ft = ft_ref[0]
    nc = nc_ref[...]
    pv = pv_ref[...]
    wf = wf_ref[...]
    wn = wn_ref[...]
    sl = sl_ref[...]
    acc = lax.dot_general(ft, wf, (((0,), (0,)), ((), ())),
                          preferred_element_type=jnp.float32)
    acc = acc + lax.dot_general(nc, wn, (((1,), (0,)), ((), ())),
                                preferred_element_type=jnp.float32)
    acc = acc + lax.dot_general(pv, sl, (((1,), (0,)), ((), ())),
                                preferred_element_type=jnp.float32)
    o0_ref[...] = acc[:, 0:128]
    o1_ref[...] = acc[:, 128:256]


def _k2(features, norm_coords, p_v_dist, wfT, wnc3, sel4):
    nj = NP // BP
    row_spec = pl.BlockSpec((BP, 128), lambda b, j: (b * nj + j, 0))
    return pl.pallas_call(
        _k2_body,
        grid=(B, nj),
        in_specs=[
            pl.BlockSpec((1, C, BP), lambda b, j: (b, 0, j)),
            pl.BlockSpec((BP, 3), lambda b, j: (b * nj + j, 0)),
            pl.BlockSpec((BP, 4), lambda b, j: (b * nj + j, 0)),
            pl.BlockSpec((C, COUT), lambda b, j: (0, 0)),
            pl.BlockSpec((3, COUT), lambda b, j: (0, 0)),
            pl.BlockSpec((4, COUT), lambda b, j: (0, 0)),
        ],
        out_specs=[row_spec, row_spec],
        out_shape=[jax.ShapeDtypeStruct((N, 128), jnp.float32)] * 2,
    )(features, norm_coords, p_v_dist, wfT, wnc3, sel4)


PBP = 1024


def _kp_body(g_ref, wq_ref, b_ref, p0_ref, p1_ref):
    g = g_ref[...]
    wq = wq_ref[...]
    bias = b_ref[...]
    inv = 1.0 / jnp.maximum(g[:, 3:4], 1.0)
    q = g * inv
    p = bias - lax.dot_general(q, wq, (((1,), (0,)), ((), ())),
                               preferred_element_type=jnp.float32)
    p0_ref[...] = p[:, 0:128]
    p1_ref[...] = p[:, 128:256]


def _kp(sums2, wq4, b2):
    row_spec = pl.BlockSpec((PBP, 128), lambda i: (i, 0))
    return pl.pallas_call(
        _kp_body,
        grid=(NSEG // PBP,),
        in_specs=[
            pl.BlockSpec((PBP, 4), lambda i: (i, 0)),
            pl.BlockSpec((4, COUT), lambda i: (0, 0)),
            pl.BlockSpec((1, COUT), lambda i: (0, 0)),
        ],
        out_specs=[row_spec, row_spec],
        out_shape=[jax.ShapeDtypeStruct((NSEG, 128), jnp.float32)] * 2,
    )(sums2, wq4, b2)


SCAN_CHUNK = 4096
N_SCAN = N // SCAN_CHUNK
SUB = 128
SEG_PER_W = NSEG // NW
HALF = SEG_PER_W // 2
LCAP = N + 256
LIST_CAP = NW * 2 * LCAP
STG = SCAN_CHUNK + 256 + 16


def _k3_body(h0, h1, idx_hbm, p0, p1,
             o0, o1, lp_hbm, ls_hbm,
             idx_v, stg_pid, stg_seg, pid_v, seg_v, rows_f, acc, cnts_v,
             lsem, sem0, sem1):
    c = lax.axis_index("c")
    s = lax.axis_index("s")
    w = s * NC + c
    iota = lax.iota(jnp.int32, 16)
    z16 = jnp.full((16,), -3.0e38, jnp.float32)
    lo = w * SEG_PER_W

    def abody(t, carry):
        cA, fA, cB, fB = carry
        pltpu.sync_copy(idx_hbm.at[pl.ds(t * SCAN_CHUNK, SCAN_CHUNK)], idx_v)

        def sbody(i, cnts):
            cA_, cB_ = cnts
            v = idx_v[pl.ds(i * 16, 16)]
            rel = v - lo
            inw = (rel >= 0) & (rel < SEG_PER_W)
            mA = inw & (rel < HALF)
            mB = inw & (rel >= HALF)
            pidv = t * SCAN_CHUNK + i * 16 + iota
            plsc.store_compressed(stg_pid.at[pl.ds(cA_, 16)], pidv, mask=mA)
            plsc.store_compressed(stg_seg.at[pl.ds(cA_, 16)], rel, mask=mA)
            plsc.store_compressed(stg_pid.at[pl.ds(STG + cB_, 16)],
                                  pidv, mask=mB)
            plsc.store_compressed(stg_seg.at[pl.ds(STG + cB_, 16)],
                                  rel - HALF, mask=mB)
            pA = plsc.all_reduce_population_count(mA)
            pB = plsc.all_reduce_population_count(mB)
            return (cA_ + lax.reduce_max(pA, (0,)),
                    cB_ + lax.reduce_max(pB, (0,)))

        cA, cB = lax.fori_loop(0, SCAN_CHUNK // 16, sbody, (cA, cB))

        out = []
        for half, (cnt, fl) in ((0, (cA, fA)), (1, (cB, fB))):
            hb = half * STG
            wbase = (w * 2 + half) * LCAP
            q = cnt >> 8
            rem = cnt - q * 256

            def flb(j, carry2):
                pltpu.sync_copy(stg_pid.at[pl.ds(hb + j * 256, 256)],
                                lp_hbm.at[pl.ds(wbase + (fl + j) * 256, 256)])
                pltpu.sync_copy(stg_seg.at[pl.ds(hb + j * 256, 256)],
                                ls_hbm.at[pl.ds(wbase + (fl + j) * 256, 256)])
                return carry2

            lax.fori_loop(0, q, flb, 0)

            nmv = jnp.where(q > 0, (rem + 15) // 16, 0)

            def mvb(i, carry2):
                stg_pid[pl.ds(hb + i * 16, 16)] = \
                    stg_pid[pl.ds(hb + q * 256 + i * 16, 16)]
                stg_seg[pl.ds(hb + i * 16, 16)] = \
                    stg_seg[pl.ds(hb + q * 256 + i * 16, 16)]
                return carry2

            lax.fori_loop(0, nmv, mvb, 0)
            out.extend([rem, fl + q])
        return tuple(out)

    cA, fA, cB, fB = lax.fori_loop(0, N_SCAN, abody, (0, 0, 0, 0))

    for half, (cnt, fl) in ((0, (cA, fA)), (1, (cB, fB))):
        hb = half * STG
        wbase = (w * 2 + half) * LCAP
        pltpu.sync_copy(stg_pid.at[pl.ds(hb, 256)],
                        lp_hbm.at[pl.ds(wbase + fl * 256, 256)])
        pltpu.sync_copy(stg_seg.at[pl.ds(hb, 256)],
                        ls_hbm.at[pl.ds(wbase + fl * 256, 256)])
        cnts_v[half] = fl * 256 + cnt

    sems = (sem0, sem1)

    def load_and_fire(h_hbm, half, m, k, par):
        wbase = (w * 2 + half) * LCAP
        pb = par * SUB
        sb = par * (SUB + 16)
        l1 = pltpu.async_copy(lp_hbm.at[pl.ds(wbase + k * SUB, SUB)],
                              pid_v.at[pl.ds(pb, SUB)], lsem)
        l2 = pltpu.async_copy(ls_hbm.at[pl.ds(wbase + k * SUB, SUB)],
                              seg_v.at[pl.ds(sb, SUB)], lsem)
        l1.wait()
        l2.wait()
        mm = m - k * SUB
        nb = (jnp.minimum(mm, SUB) + 15) // 16

        def cbody(i2, carry2):
            lanes = i2 * 16 + iota
            pv = pid_v[pl.ds(pb + i2 * 16, 16)]
            sv = seg_v[pl.ds(sb + i2 * 16, 16)]
            ok = lanes < mm
            pid_v[pl.ds(pb + i2 * 16, 16)] = jnp.where(ok, pv, 0)
            seg_v[pl.ds(sb + i2 * 16, 16)] = jnp.where(ok, sv, HALF)
            return carry2

        lax.fori_loop(0, nb, cbody, 0)

        def fire(i2, carry2):
            pv = pid_v[pl.ds(pb + i2 * 16, 16)]
            for j in range(16):
                pid = pv[j]
                pltpu.async_copy(
                    h_hbm.at[pl.ds(pid * 128, 128)],
                    rows_f.at[pl.ds((par * SUB + i2 * 16 + j) * 128, 128)],
                    sems[par])
            return carry2

        lax.fori_loop(0, nb, fire, 0)

    def drain_and_update(half, m, k, par):
        sb = par * (SUB + 16)
        mm = m - k * SUB
        nb = (jnp.minimum(mm, SUB) + 15) // 16

        def drain(i2, carry2):
            pltpu.make_async_copy(
                h0.at[pl.ds(0, 16 * 128)],
                rows_f.at[pl.ds(par * SUB * 128, 16 * 128)],
                sems[par]).wait()
            return carry2

        lax.fori_loop(0, nb, drain, 0)

        def ubody(i3, carry3):
            segl = seg_v[pl.ds(sb + i3, 16)][0]
            ab = segl * 128
            rb = (par * SUB + i3) * 128
            rs = [rows_f[pl.ds(rb + j * 16, 16)] for j in range(8)]
            avs = [acc[pl.ds(ab + j * 16, 16)] for j in range(8)]
            for j in range(8):
                acc[pl.ds(ab + j * 16, 16)] = jnp.maximum(avs[j], rs[j])
            return carry3

        lax.fori_loop(0, nb * 16, ubody, 0)

    for h_hbm, p_hbm, o_hbm in ((h0, p0, o0), (h1, p1, o1)):
        def hbody(half, _hcarry):
            def zbody(i, carry):
                acc[pl.ds(i * 16, 16)] = z16
                return carry
            lax.fori_loop(0, (HALF + 1) * 128 // 16, zbody, 0)

            m = cnts_v[half]
            nf = (m + SUB - 1) // SUB

            @pl.when(nf > 0)
            def _():
                load_and_fire(h_hbm, half, m, 0, 0)

            def kk_body(kk, carry):
                for par in range(2):
                    k = 2 * kk + par

                    @pl.when(k + 1 < nf)
                    def _():
                        load_and_fire(h_hbm, half, m, k + 1, 1 - par)

                    @pl.when(k < nf)
                    def _():
                        drain_and_update(half, m, k, par)
                return carry

            lax.fori_loop(0, (nf + 1) // 2, kk_body, 0)

            ob = (w * SEG_PER_W + half * HALF) * 128
            for sl in range(2):
                pltpu.sync_copy(p_hbm.at[pl.ds(ob + sl * 256 * 128, 256 * 128)],
                                rows_f.at[pl.ds(0, 256 * 128)])

                def rbody(i, carry):
                    a = acc[pl.ds(sl * 256 * 128 + i * 16, 16)]
                    pv_ = rows_f[pl.ds(i * 16, 16)]
                    acc[pl.ds(sl * 256 * 128 + i * 16, 16)] = \
                        jnp.maximum(a + pv_, 0.0)
                    return carry

                lax.fori_loop(0, 256 * 128 // 16, rbody, 0)
            pltpu.sync_copy(acc.at[pl.ds(0, HALF * 128)],
                            o_hbm.at[pl.ds(ob, HALF * 128)])
            return _hcarry

        lax.fori_loop(0, 2, hbody, 0)


def _k3(h0, h1, idx, p0, p1):
    mesh = plsc.VectorSubcoreMesh(core_axis_name="c", subcore_axis_name="s",
                                  num_cores=NC, num_subcores=NS)
    f = pl.kernel(
        _k3_body,
        out_type=[jax.ShapeDtypeStruct((NSEG * 128,), jnp.float32)] * 2
                 + [jax.ShapeDtypeStruct((LIST_CAP,), jnp.int32)] * 2,
        mesh=mesh,
        scratch_types=[
            pltpu.VMEM((SCAN_CHUNK,), jnp.int32),
            pltpu.VMEM((2 * STG,), jnp.int32),
            pltpu.VMEM((2 * STG,), jnp.int32),
            pltpu.VMEM((2 * SUB,), jnp.int32),
            pltpu.VMEM((2 * (SUB + 16),), jnp.int32),
            pltpu.VMEM((2 * SUB * 128,), jnp.float32),
            pltpu.VMEM(((HALF + 1) * 128,), jnp.float32),
            pltpu.SMEM((8,), jnp.int32),
            pltpu.SemaphoreType.DMA,
            pltpu.SemaphoreType.DMA,
            pltpu.SemaphoreType.DMA,
        ],
        compiler_params=pltpu.CompilerParams(needs_layout_passes=False,
                                             use_tc_tiling_on_sc=False),
    )
    return f(h0, h1, idx, p0, p1)


def kernel(features, norm_coords, coords_int, p_v_dist, proj_axis, W, b):
    base3 = jnp.arange(3)
    axes = base3 + (base3 >= proj_axis).astype(base3.dtype)
    ci = jnp.take(coords_int, axes, axis=1).astype(jnp.int32)

    ci_flat = ci.reshape(-1)
    vals_flat = jnp.concatenate(
        [norm_coords, jnp.ones((N, 1), jnp.float32)], axis=1).reshape(-1)
    zflat = jnp.zeros((NSEG * 4 // NS,), jnp.float32)
    sums_flat, idx = _k1(ci_flat, vals_flat, zflat)

    wfT = W[:, :C].T
    wnc3 = W[:, C + 2:C + 5].T
    wq4 = jnp.concatenate(
        [W[:, C + 2:C + 5].T, jnp.zeros((1, COUT), jnp.float32)], axis=0)
    arange4 = jnp.arange(4)
    sel4 = ((arange4 == axes[1]).astype(jnp.float32)[:, None]
            * W[:, C].reshape(1, COUT)
            + (arange4 == axes[2]).astype(jnp.float32)[:, None]
            * W[:, C + 1].reshape(1, COUT))
    b2 = b.reshape(1, COUT)

    p0, p1 = _kp(sums_flat.reshape(NSEG, 4), wq4, b2)
    h0, h1 = _k2(features, norm_coords, p_v_dist, wfT, wnc3, sel4)

    o0, o1, _, _ = _k3(h0.reshape(-1), h1.reshape(-1), idx,
                       p0.reshape(-1), p1.reshape(-1))
    out = jnp.concatenate(
        [o0.reshape(NSEG, 128), o1.reshape(NSEG, 128)], axis=1)
    return out.reshape(B, R, R, COUT)


# speedup vs baseline: 1.0573x; 1.0573x over previous
import jax
import jax.numpy as jnp
from jax import lax
from jax.experimental import pallas as pl
from jax.experimental.pallas import tpu as pltpu
from jax.experimental.pallas import tpu_sc as plsc

B, C, NP, R, COUT = 32, 128, 4096, 32, 256
N = B * NP
NSEG = B * R * R

NC, NS = 2, 16
NW = NC * NS

K1_CHUNK = 2048
K1_PER_TILE = N // NS
K1_NCH = K1_PER_TILE // K1_CHUNK


def _k1_body(ci_hbm, vals_hbm, z_hbm, sums_out, idxout_hbm,
             ci_v, idx_cur, vals_v, fidx_v, sums):
    c = lax.axis_index("c")
    s = lax.axis_index("s")
    pltpu.sync_copy(z_hbm, sums.at[pl.ds(s * (NSEG * 4 // NS), NSEG * 4 // NS)])
    plsc.subcore_barrier()

    iota = lax.iota(jnp.int32, 16)
    for t in range(K1_NCH):
        base = s * K1_PER_TILE + t * K1_CHUNK
        pltpu.sync_copy(ci_hbm.at[pl.ds(base * 3, K1_CHUNK * 3)], ci_v)
        pltpu.sync_copy(vals_hbm.at[pl.ds(base * 4, K1_CHUNK * 4)], vals_v)

        def body(i, carry):
            f = (i * 16 + iota) * 3
            c0 = plsc.load_gather(ci_v, [f])
            c1 = plsc.load_gather(ci_v, [f + 1])
            c2 = plsc.load_gather(ci_v, [f + 2])
            idxv = c0 * (R * R) + c1 * R + c2
            idx_cur[pl.ds(i * 16, 16)] = idxv
            return carry

        lax.fori_loop(0, K1_CHUNK // 16, body, 0)

        @pl.when(c == 0)
        def _():
            pltpu.sync_copy(idx_cur, idxout_hbm.at[pl.ds(base, K1_CHUNK)])

        def fbody(g, carry):
            pos = g * 16 + iota
            k = lax.shift_right_logical(pos, 2)
            comp = lax.bitwise_and(pos, 3)
            segs = plsc.load_gather(idx_cur, [k])
            fidx_v[pl.ds(g * 16, 16)] = segs * 4 + comp
            return carry

        lax.fori_loop(0, K1_CHUNK * 4 // 16, fbody, 0)
        pltpu.sync_copy(vals_v, sums.at[fidx_v], add=True)

    plsc.subcore_barrier()
    @pl.when(c == 0)
    def _():
        pltpu.sync_copy(sums.at[pl.ds(s * (NSEG * 4 // NS), NSEG * 4 // NS)],
                        sums_out.at[pl.ds(s * (NSEG * 4 // NS),
                                          NSEG * 4 // NS)])


def _k1(ci_flat, vals_flat, zflat):
    mesh = plsc.VectorSubcoreMesh(core_axis_name="c", subcore_axis_name="s",
                                  num_cores=NC, num_subcores=NS)
    f = pl.kernel(
        _k1_body,
        out_type=[jax.ShapeDtypeStruct((NSEG * 4,), jnp.float32),
                  jax.ShapeDtypeStruct((N,), jnp.int32)],
        mesh=mesh,
        scratch_types=[
            pltpu.VMEM((K1_CHUNK * 3,), jnp.int32),
            pltpu.VMEM((K1_CHUNK,), jnp.int32),
            pltpu.VMEM((K1_CHUNK * 4,), jnp.float32),
            pltpu.VMEM((K1_CHUNK * 4,), jnp.int32),
            pltpu.VMEM_SHARED((NSEG * 4,), jnp.float32),
        ],
        compiler_params=pltpu.CompilerParams(needs_layout_passes=False,
                                             use_tc_tiling_on_sc=False),
    )
    return f(ci_flat, vals_flat, zflat)


BP = 512


def _k2_body(ft_ref, nc_ref, pv_ref, g_ref, wf_ref, wn_ref, sl_ref,
             wq_ref, b_ref, o0_ref, o1_ref, p0_ref, p1_ref):
    ft = ft_ref[0]
    nc = nc_ref[...]
    pv = pv_ref[...]
    wf = wf_ref[...]
    wn = wn_ref[...]
    sl = sl_ref[...]
    acc = lax.dot_general(ft, wf, (((0,), (0,)), ((), ())),
                          preferred_element_type=jnp.float32)
    acc = acc + lax.dot_general(nc, wn, (((1,), (0,)), ((), ())),
                                preferred_element_type=jnp.float32)
    acc = acc + lax.dot_general(pv, sl, (((1,), (0,)), ((), ())),
                                preferred_element_type=jnp.float32)
    o0_ref[...] = acc[:, 0:128]
    o1_ref[...] = acc[:, 128:256]
    g = g_ref[...]
    inv = 1.0 / jnp.maximum(g[:, 3:4], 1.0)
    q = g * inv
    p = b_ref[...] - lax.dot_general(q, wq_ref[...], (((1,), (0,)), ((), ())),
                                     preferred_element_type=jnp.float32)
    p0_ref[...] = p[:, 0:128]
    p1_ref[...] = p[:, 128:256]


def _k2(features, norm_coords, p_v_dist, sums2, wfT, wnc3, sel4, wq4, b2):
    nj = NP // BP
    row_spec = pl.BlockSpec((BP, 128), lambda b, j: (b * nj + j, 0))
    p_spec = pl.BlockSpec((PBP, 128), lambda b, j: (b, 0))
    return pl.pallas_call(
        _k2_body,
        grid=(B, nj),
        in_specs=[
            pl.BlockSpec((1, C, BP), lambda b, j: (b, 0, j)),
            pl.BlockSpec((BP, 3), lambda b, j: (b * nj + j, 0)),
            pl.BlockSpec((BP, 4), lambda b, j: (b * nj + j, 0)),
            pl.BlockSpec((PBP, 4), lambda b, j: (b, 0)),
            pl.BlockSpec((C, COUT), lambda b, j: (0, 0)),
            pl.BlockSpec((3, COUT), lambda b, j: (0, 0)),
            pl.BlockSpec((4, COUT), lambda b, j: (0, 0)),
            pl.BlockSpec((4, COUT), lambda b, j: (0, 0)),
            pl.BlockSpec((1, COUT), lambda b, j: (0, 0)),
        ],
        out_specs=[row_spec, row_spec, p_spec, p_spec],
        out_shape=[jax.ShapeDtypeStruct((N, 128), jnp.float32)] * 2
                  + [jax.ShapeDtypeStruct((NSEG, 128), jnp.float32)] * 2,
    )(features, norm_coords, p_v_dist, sums2, wfT, wnc3, sel4, wq4, b2)


PBP = NSEG // B


SCAN_CHUNK = 4096
N_SCAN = N // SCAN_CHUNK
SUB = 128
SEG_PER_W = NSEG // NW
HALF = SEG_PER_W // 2
LCAP = N + 256
LIST_CAP = NW * 2 * LCAP
STG = SCAN_CHUNK + 256 + 16


def _k3_body(h0, h1, idx_hbm, p0, p1,
             o0, o1, lp_hbm, ls_hbm,
             idx_v, stg_pid, stg_seg, pid_v, seg_v, rows_f, acc, cnts_v,
             lsem, sem0, sem1):
    c = lax.axis_index("c")
    s = lax.axis_index("s")
    w = s * NC + c
    iota = lax.iota(jnp.int32, 16)
    z16 = jnp.full((16,), -3.0e38, jnp.float32)
    lo = w * SEG_PER_W

    def abody(t, carry):
        cA, fA, cB, fB = carry
        pltpu.sync_copy(idx_hbm.at[pl.ds(t * SCAN_CHUNK, SCAN_CHUNK)], idx_v)

        def sbody(i, cnts):
            cA_, cB_ = cnts
            v = idx_v[pl.ds(i * 16, 16)]
            rel = v - lo
            inw = (rel >= 0) & (rel < SEG_PER_W)
            mA = inw & (rel < HALF)
            mB = inw & (rel >= HALF)
            pidv = t * SCAN_CHUNK + i * 16 + iota
            plsc.store_compressed(stg_pid.at[pl.ds(cA_, 16)], pidv, mask=mA)
            plsc.store_compressed(stg_seg.at[pl.ds(cA_, 16)], rel, mask=mA)
            plsc.store_compressed(stg_pid.at[pl.ds(STG + cB_, 16)],
                                  pidv, mask=mB)
            plsc.store_compressed(stg_seg.at[pl.ds(STG + cB_, 16)],
                                  rel - HALF, mask=mB)
            pA = plsc.all_reduce_population_count(mA)
            pB = plsc.all_reduce_population_count(mB)
            return (cA_ + lax.reduce_max(pA, (0,)),
                    cB_ + lax.reduce_max(pB, (0,)))

        cA, cB = lax.fori_loop(0, SCAN_CHUNK // 16, sbody, (cA, cB))

        out = []
        for half, (cnt, fl) in ((0, (cA, fA)), (1, (cB, fB))):
            hb = half * STG
            wbase = (w * 2 + half) * LCAP
            q = cnt >> 8
            rem = cnt - q * 256

            def flb(j, carry2):
                pltpu.sync_copy(stg_pid.at[pl.ds(hb + j * 256, 256)],
                                lp_hbm.at[pl.ds(wbase + (fl + j) * 256, 256)])
                pltpu.sync_copy(stg_seg.at[pl.ds(hb + j * 256, 256)],
                                ls_hbm.at[pl.ds(wbase + (fl + j) * 256, 256)])
                return carry2

            lax.fori_loop(0, q, flb, 0)

            nmv = jnp.where(q > 0, (rem + 15) // 16, 0)

            def mvb(i, carry2):
                stg_pid[pl.ds(hb + i * 16, 16)] = \
                    stg_pid[pl.ds(hb + q * 256 + i * 16, 16)]
                stg_seg[pl.ds(hb + i * 16, 16)] = \
                    stg_seg[pl.ds(hb + q * 256 + i * 16, 16)]
                return carry2

            lax.fori_loop(0, nmv, mvb, 0)
            out.extend([rem, fl + q])
        return tuple(out)

    cA, fA, cB, fB = lax.fori_loop(0, N_SCAN, abody, (0, 0, 0, 0))

    for half, (cnt, fl) in ((0, (cA, fA)), (1, (cB, fB))):
        hb = half * STG
        wbase = (w * 2 + half) * LCAP
        pltpu.sync_copy(stg_pid.at[pl.ds(hb, 256)],
                        lp_hbm.at[pl.ds(wbase + fl * 256, 256)])
        pltpu.sync_copy(stg_seg.at[pl.ds(hb, 256)],
                        ls_hbm.at[pl.ds(wbase + fl * 256, 256)])
        cnts_v[half] = fl * 256 + cnt

    sems = (sem0, sem1)

    def load_and_fire(h_hbm, half, m, k, par):
        wbase = (w * 2 + half) * LCAP
        pb = par * SUB
        sb = par * (SUB + 16)
        l1 = pltpu.async_copy(lp_hbm.at[pl.ds(wbase + k * SUB, SUB)],
                              pid_v.at[pl.ds(pb, SUB)], lsem)
        l2 = pltpu.async_copy(ls_hbm.at[pl.ds(wbase + k * SUB, SUB)],
                              seg_v.at[pl.ds(sb, SUB)], lsem)
        l1.wait()
        l2.wait()
        mm = m - k * SUB
        nb = (jnp.minimum(mm, SUB) + 15) // 16

        def cbody(i2, carry2):
            lanes = i2 * 16 + iota
            pv = pid_v[pl.ds(pb + i2 * 16, 16)]
            sv = seg_v[pl.ds(sb + i2 * 16, 16)]
            ok = lanes < mm
            pid_v[pl.ds(pb + i2 * 16, 16)] = jnp.where(ok, pv, 0)
            seg_v[pl.ds(sb + i2 * 16, 16)] = jnp.where(ok, sv, HALF)
            return carry2

        lax.fori_loop(0, nb, cbody, 0)

        def fire(i2, carry2):
            pv = pid_v[pl.ds(pb + i2 * 16, 16)]
            for j in range(16):
                pid = pv[j]
                pltpu.async_copy(
                    h_hbm.at[pl.ds(pid * 128, 128)],
                    rows_f.at[pl.ds((par * SUB + i2 * 16 + j) * 128, 128)],
                    sems[par])
            return carry2

        lax.fori_loop(0, nb, fire, 0)

    def drain_and_update(half, m, k, par):
        sb = par * (SUB + 16)
        mm = m - k * SUB
        nb = (jnp.minimum(mm, SUB) + 15) // 16

        def drain(i2, carry2):
            pltpu.make_async_copy(
                h0.at[pl.ds(0, 16 * 128)],
                rows_f.at[pl.ds(par * SUB * 128, 16 * 128)],
                sems[par]).wait()
            return carry2

        lax.fori_loop(0, nb, drain, 0)

        def ubody(i3, carry3):
            segl = seg_v[pl.ds(sb + i3, 16)][0]
            ab = segl * 128
            rb = (par * SUB + i3) * 128
            rs = [rows_f[pl.ds(rb + j * 16, 16)] for j in range(8)]
            avs = [acc[pl.ds(ab + j * 16, 16)] for j in range(8)]
            for j in range(8):
                acc[pl.ds(ab + j * 16, 16)] = jnp.maximum(avs[j], rs[j])
            return carry3

        lax.fori_loop(0, nb * 16, ubody, 0)

    for h_hbm, p_hbm, o_hbm in ((h0, p0, o0), (h1, p1, o1)):
        def hbody(half, _hcarry):
            def zbody(i, carry):
                acc[pl.ds(i * 16, 16)] = z16
                return carry
            lax.fori_loop(0, (HALF + 1) * 128 // 16, zbody, 0)

            m = cnts_v[half]
            nf = (m + SUB - 1) // SUB

            @pl.when(nf > 0)
            def _():
                load_and_fire(h_hbm, half, m, 0, 0)

            def kk_body(kk, carry):
                for par in range(2):
                    k = 2 * kk + par

                    @pl.when(k + 1 < nf)
                    def _():
                        load_and_fire(h_hbm, half, m, k + 1, 1 - par)

                    @pl.when(k < nf)
                    def _():
                        drain_and_update(half, m, k, par)
                return carry

            lax.fori_loop(0, (nf + 1) // 2, kk_body, 0)

            ob = (w * SEG_PER_W + half * HALF) * 128
            for sl in range(2):
                pltpu.sync_copy(p_hbm.at[pl.ds(ob + sl * 256 * 128, 256 * 128)],
                                rows_f.at[pl.ds(0, 256 * 128)])

                def rbody(i, carry):
                    base = sl * 256 * 128 + i * 64
                    avs = [acc[pl.ds(base + j * 16, 16)] for j in range(4)]
                    pvs = [rows_f[pl.ds(i * 64 + j * 16, 16)]
                           for j in range(4)]
                    for j in range(4):
                        acc[pl.ds(base + j * 16, 16)] = \
                            jnp.maximum(avs[j] + pvs[j], 0.0)
                    return carry

                lax.fori_loop(0, 256 * 128 // 64, rbody, 0)
            pltpu.sync_copy(acc.at[pl.ds(0, HALF * 128)],
                            o_hbm.at[pl.ds(ob, HALF * 128)])
            return _hcarry

        lax.fori_loop(0, 2, hbody, 0)


def _k3(h0, h1, idx, p0, p1):
    mesh = plsc.VectorSubcoreMesh(core_axis_name="c", subcore_axis_name="s",
                                  num_cores=NC, num_subcores=NS)
    f = pl.kernel(
        _k3_body,
        out_type=[jax.ShapeDtypeStruct((NSEG * 128,), jnp.float32)] * 2
                 + [jax.ShapeDtypeStruct((LIST_CAP,), jnp.int32)] * 2,
        mesh=mesh,
        scratch_types=[
            pltpu.VMEM((SCAN_CHUNK,), jnp.int32),
            pltpu.VMEM((2 * STG,), jnp.int32),
            pltpu.VMEM((2 * STG,), jnp.int32),
            pltpu.VMEM((2 * SUB,), jnp.int32),
            pltpu.VMEM((2 * (SUB + 16),), jnp.int32),
            pltpu.VMEM((2 * SUB * 128,), jnp.float32),
            pltpu.VMEM(((HALF + 1) * 128,), jnp.float32),
            pltpu.SMEM((8,), jnp.int32),
            pltpu.SemaphoreType.DMA,
            pltpu.SemaphoreType.DMA,
            pltpu.SemaphoreType.DMA,
        ],
        compiler_params=pltpu.CompilerParams(needs_layout_passes=False,
                                             use_tc_tiling_on_sc=False),
    )
    return f(h0, h1, idx, p0, p1)


def kernel(features, norm_coords, coords_int, p_v_dist, proj_axis, W, b):
    base3 = jnp.arange(3)
    axes = base3 + (base3 >= proj_axis).astype(base3.dtype)
    ci = jnp.take(coords_int, axes, axis=1).astype(jnp.int32)

    ci_flat = ci.reshape(-1)
    vals_flat = jnp.concatenate(
        [norm_coords, jnp.ones((N, 1), jnp.float32)], axis=1).reshape(-1)
    zflat = jnp.zeros((NSEG * 4 // NS,), jnp.float32)
    sums_flat, idx = _k1(ci_flat, vals_flat, zflat)

    wfT = W[:, :C].T
    wnc3 = W[:, C + 2:C + 5].T
    wq4 = jnp.concatenate(
        [W[:, C + 2:C + 5].T, jnp.zeros((1, COUT), jnp.float32)], axis=0)
    arange4 = jnp.arange(4)
    sel4 = ((arange4 == axes[1]).astype(jnp.float32)[:, None]
            * W[:, C].reshape(1, COUT)
            + (arange4 == axes[2]).astype(jnp.float32)[:, None]
            * W[:, C + 1].reshape(1, COUT))
    b2 = b.reshape(1, COUT)

    h0, h1, p0, p1 = _k2(features, norm_coords, p_v_dist,
                         sums_flat.reshape(NSEG, 4), wfT, wnc3, sel4,
                         wq4, b2)

    o0, o1, _, _ = _k3(h0.reshape(-1), h1.reshape(-1), idx,
                       p0.reshape(-1), p1.reshape(-1))
    out = jnp.concatenate(
        [o0.reshape(NSEG, 128), o1.reshape(NSEG, 128)], axis=1)
    return out.reshape(B, R, R, COUT)
